# baseline (speedup 1.0000x reference)
import math

import jax
import jax.numpy as jnp
import numpy as np
from jax import lax
from jax.experimental import pallas as pl
from jax.experimental.pallas import tpu as pltpu
from jax.experimental.pallas import tpu_sc as plsc

VOCAB = 1000000
D = 64
B = 4096
L = 200
N = B * L
NW = 32
NPW = N // NW
C = 128
NCH = NPW // C
NB16 = C // 16
H = C // 2
PEROWS = L + C
PELINES = (PEROWS // 2 + 7) // 8 * 8


def _make_pe():
    pe = np.zeros((PEROWS, D), dtype=np.float32)
    position = np.arange(0.0, PEROWS, dtype=np.float64)[:, None] % L
    div_term = np.exp(
        np.arange(0.0, D, 2, dtype=np.float64) * -(math.log(10000.0) / D))
    pe[:, 0::2] = np.sin(position * div_term)
    pe[:, 1::2] = np.cos(position * div_term)
    out = np.zeros((PELINES, 2 * D), dtype=np.float32)
    out.reshape(-1)[: PEROWS * D] = pe.reshape(-1)
    return out


_PE_PACKED = _make_pe()

_SCALE = math.sqrt(D)


def _emb_body(x_hbm, pe_hbm, lut_hbm, out_hbm,
              idx_v, pe_v, r0, r1, o0, o1, x0, x1,
              g0, g1, s0, s1):
    rows = [r0, r1]
    outc = [o0, o1]
    gidx = [x0, x1]
    gsem = [g0, g1]
    osem = [s0, s1]

    cid = lax.axis_index("c")
    sid = lax.axis_index("s")
    wid = sid * 2 + cid
    obase = wid * NPW

    pltpu.sync_copy(x_hbm.at[wid], idx_v)
    pltpu.sync_copy(pe_hbm, pe_v)

    def make_gidx(cc, slot):
        def blk(b16, carry):
            sl = pl.ds(b16 * 16, 16)
            xv = idx_v[cc, sl]
            gidx[slot][sl] = xv - jnp.where(
                xv >= VOCAB // 2, VOCAB // 2, 0).astype(jnp.int32)
            return carry
        lax.fori_loop(0, NB16, blk, 0)

    def issue_gather(slot):
        pltpu.async_copy(
            lut_hbm.at[gidx[slot].at[pl.ds(0, H)]],
            rows[slot].at[pl.ds(0, H)], gsem[slot])
        pltpu.async_copy(
            lut_hbm.at[gidx[slot].at[pl.ds(H, H)]],
            rows[slot].at[pl.ds(H, H)], gsem[slot])

    def wait_gather(slot):
        pltpu.make_async_copy(
            lut_hbm.at[gidx[slot].at[pl.ds(0, H)]],
            rows[slot].at[pl.ds(0, H)], gsem[slot]).wait()
        pltpu.make_async_copy(
            lut_hbm.at[gidx[slot].at[pl.ds(H, H)]],
            rows[slot].at[pl.ds(H, H)], gsem[slot]).wait()

    make_gidx(0, 0)
    issue_gather(0)

    def compute(c, rv, ov):
        ph = lax.rem(c * C, L)
        phh = lax.shift_right_logical(ph, 1)

        def blk(b16, carry2):
            jb = b16 * 16
            sl = pl.ds(jb, 16)
            off16 = jnp.where(idx_v[c, sl] >= VOCAB // 2, D, 0).astype(
                jnp.int32)
            for jj in range(16):
                j = jb + jj
                off = off16[jj]
                prow = phh + b16 * 8 + (jj >> 1)
                pc0 = (jj & 1) * D
                for d in range(D // 16):
                    ov[j, pl.ds(d * 16, 16)] = (
                        rv[j, pl.ds(off + d * 16, 16)] * _SCALE
                        + pe_v[prow, pl.ds(pc0 + d * 16, 16)])
            return carry2

        lax.fori_loop(0, NB16, blk, 0)

    def chunk(c, carry):
        r = lax.rem(c, 2)

        @pl.when(r == 0)
        def _():
            wait_gather(0)

            @pl.when(c < NCH - 1)
            def _():
                make_gidx(c + 1, 1)
                issue_gather(1)

            @pl.when(c >= 2)
            def _():
                pltpu.make_async_copy(
                    outc[0], out_hbm.at[pl.ds(obase, C)], osem[0]).wait()

            compute(c, rows[0], outc[0])
            pltpu.async_copy(
                outc[0], out_hbm.at[pl.ds(obase + c * C, C)], osem[0])

        @pl.when(r == 1)
        def _():
            wait_gather(1)

            @pl.when(c < NCH - 1)
            def _():
                make_gidx(c + 1, 0)
                issue_gather(0)

            @pl.when(c >= 2)
            def _():
                pltpu.make_async_copy(
                    outc[1], out_hbm.at[pl.ds(obase, C)], osem[1]).wait()

            compute(c, rows[1], outc[1])
            pltpu.async_copy(
                outc[1], out_hbm.at[pl.ds(obase + c * C, C)], osem[1])

        return carry

    lax.fori_loop(0, NCH, chunk, 0)

    for b in range(2):
        pltpu.make_async_copy(
            outc[b], out_hbm.at[pl.ds(obase, C)], osem[b]).wait()


_emb_call = pl.kernel(
    _emb_body,
    out_type=jax.ShapeDtypeStruct((N, D), jnp.float32),
    mesh=plsc.VectorSubcoreMesh(core_axis_name="c", subcore_axis_name="s"),
    scratch_types=(
        [pltpu.VMEM((NCH, C), jnp.int32),
         pltpu.VMEM((PELINES, 2 * D), jnp.float32)]
        + [pltpu.VMEM((C, 2 * D), jnp.float32) for _ in range(2)]
        + [pltpu.VMEM((C, D), jnp.float32) for _ in range(2)]
        + [pltpu.VMEM((C,), jnp.int32) for _ in range(2)]
        + [pltpu.SemaphoreType.DMA for _ in range(4)]
    ),
)


def _compact_body(i1_ref, i2_ref, o_ref):
    o_ref[...] = jnp.concatenate([i1_ref[...], i2_ref[...]], axis=1)


_CBLK = 1000

_compact_call = pl.pallas_call(
    _compact_body,
    grid=(VOCAB // 2 // _CBLK,),
    in_specs=[
        pl.BlockSpec((_CBLK, D), lambda k: (k, 0)),
        pl.BlockSpec((_CBLK, D), lambda k: (k + VOCAB // 2 // _CBLK, 0)),
    ],
    out_specs=pl.BlockSpec((_CBLK, 2 * D), lambda k: (k, 0)),
    out_shape=jax.ShapeDtypeStruct((VOCAB // 2, 2 * D), jnp.float32),
)


def kernel(x, lut):
    xr = x.reshape(NW, NCH, C).astype(jnp.int32)
    lut2 = _compact_call(lut, lut)
    pe = jnp.asarray(_PE_PACKED)
    out = _emb_call(xr, pe, lut2)
    return out.reshape(B, L, D)


# speedup vs baseline: 1.0287x; 1.0287x over previous
import math

import jax
import jax.numpy as jnp
import numpy as np
from jax import lax
from jax.experimental import pallas as pl
from jax.experimental.pallas import tpu as pltpu
from jax.experimental.pallas import tpu_sc as plsc

VOCAB = 1000000
D = 64
B = 4096
L = 200
N = B * L
NW = 32
NPW = N // NW
C = 128
NCH = NPW // C
NB16 = C // 16
H = C // 2
PEROWS = L + C
PELINES = (PEROWS // 2 + 7) // 8 * 8


def _make_pe():
    pe = np.zeros((PEROWS, D), dtype=np.float32)
    position = np.arange(0.0, PEROWS, dtype=np.float64)[:, None] % L
    div_term = np.exp(
        np.arange(0.0, D, 2, dtype=np.float64) * -(math.log(10000.0) / D))
    pe[:, 0::2] = np.sin(position * div_term)
    pe[:, 1::2] = np.cos(position * div_term)
    out = np.zeros((PELINES, 2 * D), dtype=np.float32)
    out.reshape(-1)[: PEROWS * D] = pe.reshape(-1)
    return out


_PE_PACKED = _make_pe()

_SCALE = math.sqrt(D)


def _emb_body(x_hbm, pe_hbm, lut_hbm, out_hbm,
              idx_v, pe_v, r0, r1, o0, o1, x0, x1,
              g0, g1, s0, s1):
    rows = [r0, r1]
    outc = [o0, o1]
    gidx = [x0, x1]
    gsem = [g0, g1]
    osem = [s0, s1]

    cid = lax.axis_index("c")
    sid = lax.axis_index("s")
    wid = sid * 2 + cid
    obase = wid * NPW

    pltpu.sync_copy(x_hbm.at[wid], idx_v)
    pltpu.sync_copy(pe_hbm, pe_v)

    def make_gidx(cc, slot):
        def blk(b16, carry):
            sl = pl.ds(b16 * 16, 16)
            xv = idx_v[cc, sl]
            gidx[slot][sl] = xv - jnp.where(
                xv >= VOCAB // 2, VOCAB // 2, 0).astype(jnp.int32)
            return carry
        lax.fori_loop(0, NB16, blk, 0)

    def issue_gather(slot):
        pltpu.async_copy(
            lut_hbm.at[gidx[slot].at[pl.ds(0, H)]],
            rows[slot].at[pl.ds(0, H)], gsem[slot])
        pltpu.async_copy(
            lut_hbm.at[gidx[slot].at[pl.ds(H, H)]],
            rows[slot].at[pl.ds(H, H)], gsem[slot])

    def wait_gather(slot):
        pltpu.make_async_copy(
            lut_hbm.at[gidx[slot].at[pl.ds(0, H)]],
            rows[slot].at[pl.ds(0, H)], gsem[slot]).wait()
        pltpu.make_async_copy(
            lut_hbm.at[gidx[slot].at[pl.ds(H, H)]],
            rows[slot].at[pl.ds(H, H)], gsem[slot]).wait()

    make_gidx(0, 0)
    issue_gather(0)

    def compute(c, rv, ov):
        ph = lax.rem(c * C, L)
        phh = lax.shift_right_logical(ph, 1)

        def blk(b16, carry2):
            jb = b16 * 16
            sl = pl.ds(jb, 16)
            off16 = jnp.where(idx_v[c, sl] >= VOCAB // 2, D, 0).astype(
                jnp.int32)
            for jj in range(16):
                j = jb + jj
                off = off16[jj]
                prow = phh + b16 * 8 + (jj >> 1)
                pc0 = (jj & 1) * D
                for d in range(D // 16):
                    ov[j, pl.ds(d * 16, 16)] = (
                        rv[j, pl.ds(off + d * 16, 16)] * _SCALE
                        + pe_v[prow, pl.ds(pc0 + d * 16, 16)])
            return carry2

        lax.fori_loop(0, NB16, blk, 0)

    def chunk(c, carry):
        r = lax.rem(c, 2)

        @pl.when(r == 0)
        def _():
            wait_gather(0)

            @pl.when(c < NCH - 1)
            def _():
                make_gidx(c + 1, 1)
                issue_gather(1)

            @pl.when(c >= 2)
            def _():
                pltpu.make_async_copy(
                    outc[0], out_hbm.at[pl.ds(obase, C)], osem[0]).wait()

            compute(c, rows[0], outc[0])
            pltpu.async_copy(
                outc[0], out_hbm.at[pl.ds(obase + c * C, C)], osem[0])

        @pl.when(r == 1)
        def _():
            wait_gather(1)

            @pl.when(c < NCH - 1)
            def _():
                make_gidx(c + 1, 0)
                issue_gather(0)

            @pl.when(c >= 2)
            def _():
                pltpu.make_async_copy(
                    outc[1], out_hbm.at[pl.ds(obase, C)], osem[1]).wait()

            compute(c, rows[1], outc[1])
            pltpu.async_copy(
                outc[1], out_hbm.at[pl.ds(obase + c * C, C)], osem[1])

        return carry

    lax.fori_loop(0, NCH, chunk, 0)

    for b in range(2):
        pltpu.make_async_copy(
            outc[b], out_hbm.at[pl.ds(obase, C)], osem[b]).wait()


_emb_call = pl.kernel(
    _emb_body,
    out_type=jax.ShapeDtypeStruct((N, D), jnp.float32),
    mesh=plsc.VectorSubcoreMesh(core_axis_name="c", subcore_axis_name="s"),
    scratch_types=(
        [pltpu.VMEM((NCH, C), jnp.int32),
         pltpu.VMEM((PELINES, 2 * D), jnp.float32)]
        + [pltpu.VMEM((C, 2 * D), jnp.float32) for _ in range(2)]
        + [pltpu.VMEM((C, D), jnp.float32) for _ in range(2)]
        + [pltpu.VMEM((C,), jnp.int32) for _ in range(2)]
        + [pltpu.SemaphoreType.DMA for _ in range(4)]
    ),
)


def _compact_body(i1_ref, i2_ref, o_ref):
    o_ref[...] = jnp.concatenate([i1_ref[...], i2_ref[...]], axis=1)


_CBLK = 1000

_compact_call = pl.pallas_call(
    _compact_body,
    grid=(VOCAB // 2 // _CBLK,),
    in_specs=[
        pl.BlockSpec((_CBLK, D), lambda k: (k, 0)),
        pl.BlockSpec((_CBLK, D), lambda k: (k + VOCAB // 2 // _CBLK, 0)),
    ],
    out_specs=pl.BlockSpec((_CBLK, 2 * D), lambda k: (k, 0)),
    out_shape=jax.ShapeDtypeStruct((VOCAB // 2, 2 * D), jnp.float32),
)


def kernel(x, lut):
    xr = x.reshape(NW, NCH, C).astype(jnp.int32)
    lut2 = jnp.concatenate(
        [lut[: VOCAB // 2], lut[VOCAB // 2:]], axis=1)
    pe = jnp.asarray(_PE_PACKED)
    out = _emb_call(xr, pe, lut2)
    return out.reshape(B, L, D)


# speedup vs baseline: 1.1302x; 1.0987x over previous
import math

import jax
import jax.numpy as jnp
import numpy as np
from jax import lax
from jax.experimental import pallas as pl
from jax.experimental.pallas import tpu as pltpu
from jax.experimental.pallas import tpu_sc as plsc

VOCAB = 1000000
D = 64
B = 4096
L = 200
N = B * L
NW = 32
NPW = N // NW
C = 128
NCH = NPW // C
NB16 = C // 16
H = C // 2
PEROWS = L + C
PELINES = (PEROWS // 2 + 7) // 8 * 8


def _make_pe():
    pe = np.zeros((PEROWS, D), dtype=np.float32)
    position = np.arange(0.0, PEROWS, dtype=np.float64)[:, None] % L
    div_term = np.exp(
        np.arange(0.0, D, 2, dtype=np.float64) * -(math.log(10000.0) / D))
    pe[:, 0::2] = np.sin(position * div_term)
    pe[:, 1::2] = np.cos(position * div_term)
    out = np.zeros((PELINES, 2 * D), dtype=np.float32)
    out.reshape(-1)[: PEROWS * D] = pe.reshape(-1)
    return out


_PE_PACKED = _make_pe()

_SCALE = math.sqrt(D)


def _emb_body(x_hbm, pe_hbm, lut_hbm, out_hbm,
              idx_v, pe_v, r0, r1, o0, o1, x0, x1,
              g0, g1, s0, s1):
    rows = [r0, r1]
    outc = [o0, o1]
    gidx = [x0, x1]
    gsem = [g0, g1]
    osem = [s0, s1]

    cid = lax.axis_index("c")
    sid = lax.axis_index("s")
    wid = sid * 2 + cid
    obase = wid * NPW

    pltpu.sync_copy(x_hbm.at[wid], idx_v)
    pltpu.sync_copy(pe_hbm, pe_v)

    def make_gidx(cc, slot):
        def blk(b16, carry):
            sl = pl.ds(b16 * 16, 16)
            gidx[slot][sl] = lax.shift_right_logical(idx_v[cc, sl], 1)
            return carry
        lax.fori_loop(0, NB16, blk, 0)

    def issue_gather(slot):
        pltpu.async_copy(
            lut_hbm.at[gidx[slot].at[pl.ds(0, H)]],
            rows[slot].at[pl.ds(0, H)], gsem[slot])
        pltpu.async_copy(
            lut_hbm.at[gidx[slot].at[pl.ds(H, H)]],
            rows[slot].at[pl.ds(H, H)], gsem[slot])

    def wait_gather(slot):
        pltpu.make_async_copy(
            lut_hbm.at[gidx[slot].at[pl.ds(0, H)]],
            rows[slot].at[pl.ds(0, H)], gsem[slot]).wait()
        pltpu.make_async_copy(
            lut_hbm.at[gidx[slot].at[pl.ds(H, H)]],
            rows[slot].at[pl.ds(H, H)], gsem[slot]).wait()

    make_gidx(0, 0)
    issue_gather(0)

    def compute(c, rv, ov):
        ph = lax.rem(c * C, L)
        phh = lax.shift_right_logical(ph, 1)

        def blk(b16, carry2):
            jb = b16 * 16
            sl = pl.ds(jb, 16)
            off16 = (idx_v[c, sl] & 1) * D
            for jj in range(16):
                j = jb + jj
                off = off16[jj]
                prow = phh + b16 * 8 + (jj >> 1)
                pc0 = (jj & 1) * D
                for d in range(D // 16):
                    ov[j, pl.ds(d * 16, 16)] = (
                        rv[j, pl.ds(off + d * 16, 16)] * _SCALE
                        + pe_v[prow, pl.ds(pc0 + d * 16, 16)])
            return carry2

        lax.fori_loop(0, NB16, blk, 0)

    def chunk(c, carry):
        r = lax.rem(c, 2)

        @pl.when(r == 0)
        def _():
            @pl.when(c < NCH - 1)
            def _():
                make_gidx(c + 1, 1)
                issue_gather(1)

            wait_gather(0)

            @pl.when(c >= 2)
            def _():
                pltpu.make_async_copy(
                    outc[0], out_hbm.at[pl.ds(obase, C)], osem[0]).wait()

            compute(c, rows[0], outc[0])
            pltpu.async_copy(
                outc[0], out_hbm.at[pl.ds(obase + c * C, C)], osem[0])

        @pl.when(r == 1)
        def _():
            @pl.when(c < NCH - 1)
            def _():
                make_gidx(c + 1, 0)
                issue_gather(0)

            wait_gather(1)

            @pl.when(c >= 2)
            def _():
                pltpu.make_async_copy(
                    outc[1], out_hbm.at[pl.ds(obase, C)], osem[1]).wait()

            compute(c, rows[1], outc[1])
            pltpu.async_copy(
                outc[1], out_hbm.at[pl.ds(obase + c * C, C)], osem[1])

        return carry

    lax.fori_loop(0, NCH, chunk, 0)

    for b in range(2):
        pltpu.make_async_copy(
            outc[b], out_hbm.at[pl.ds(obase, C)], osem[b]).wait()


_emb_call = pl.kernel(
    _emb_body,
    out_type=jax.ShapeDtypeStruct((N, D), jnp.float32),
    mesh=plsc.VectorSubcoreMesh(core_axis_name="c", subcore_axis_name="s"),
    scratch_types=(
        [pltpu.VMEM((NCH, C), jnp.int32),
         pltpu.VMEM((PELINES, 2 * D), jnp.float32)]
        + [pltpu.VMEM((C, 2 * D), jnp.float32) for _ in range(2)]
        + [pltpu.VMEM((C, D), jnp.float32) for _ in range(2)]
        + [pltpu.VMEM((C,), jnp.int32) for _ in range(2)]
        + [pltpu.SemaphoreType.DMA for _ in range(4)]
    ),
)


def kernel(x, lut):
    xr = x.reshape(NW, NCH, C).astype(jnp.int32)
    lut2 = lut.reshape(VOCAB // 2, 2 * D)
    pe = jnp.asarray(_PE_PACKED)
    out = _emb_call(xr, pe, lut2)
    return out.reshape(B, L, D)
